# baseline (speedup 1.0000x reference)
import jax
import jax.numpy as jnp
from jax.experimental import pallas as pl
from jax.experimental.pallas import tpu as pltpu
import math

_W = 0.1
_NBINS = 64
_EPS = 1e-08
_LOG2E = 1.4426950408889634
_K = 8
_CH = 64


def _sum8(v):
    return jnp.sum(v.reshape(_CH // 8, 8, 128), axis=0)


def _one_batch(t_ref, p_ref, i, out_ref, hist_p, hist_t, zt_ref, zp_ref):
    t = t_ref[...]
    rows = t.shape[0]
    nch = rows // _CH

    vmin = jnp.min(t)
    vmax = jnp.max(t)
    denom = vmax - vmin + _EPS
    ia = _NBINS / denom
    zt_ref[...] = (t_ref[...] - vmin) * ia
    zp_ref[...] = (p_ref[...] - vmin) * ia

    nL2 = jnp.float32(-_LOG2E / 2.0)
    L = jnp.float32(_LOG2E)
    G = _NBINS // _K
    scales = [jnp.float32(math.exp(-(k * k) / 2.0)) for k in range(_K)]

    for g in range(G):
        sh = jnp.float32(g * _K + 0.5)
        lo = jnp.float32(-40.0)
        hi = jnp.float32(_K + 39.0)

        def chunk_body(c, accs, z_ref=None):
            zc = z_ref[pl.ds(c * _CH, _CH), :]
            u = jnp.minimum(jnp.maximum(zc - sh, lo), hi)
            S = jnp.exp2((u * nL2) * u)
            r = jnp.exp2(u * L)
            out = []
            for k in range(_K):
                if k:
                    S = S * r
                out.append(accs[k] + _sum8(S))
            return tuple(out)

        zero = tuple(jnp.zeros((8, 128), jnp.float32) for _ in range(_K))

        acc_t = jax.lax.fori_loop(
            0, nch, lambda c, a: chunk_body(c, a, z_ref=zt_ref), zero,
            unroll=8)
        acc_p = jax.lax.fori_loop(
            0, nch, lambda c, a: chunk_body(c, a, z_ref=zp_ref), zero,
            unroll=8)
        for k in range(_K):
            b = g * _K + k
            hist_t[b:b + 1, :] = scales[k] * jnp.sum(
                acc_t[k], axis=0, keepdims=True)
            hist_p[b:b + 1, :] = scales[k] * jnp.sum(
                acc_p[k], axis=0, keepdims=True)

    ht = jnp.sum(hist_t[...], axis=1, keepdims=True)
    hp = jnp.sum(hist_p[...], axis=1, keepdims=True)
    tp = ht / (jnp.sum(ht) + _EPS)
    pp = hp / (jnp.sum(hp) + _EPS)
    kl = jnp.sum(tp * (jnp.log(tp + _EPS) - jnp.log(pp + _EPS)))
    out_ref[i] = jnp.full((8, 128), kl, dtype=jnp.float32)


def _kl_body(p_hbm, t_hbm, out_ref,
             buf_t, buf_p, hist_p, hist_t, zt_ref, zp_ref, sem_t, sem_p):
    nb = p_hbm.shape[0]

    def dma_in(slot, i):
        pltpu.make_async_copy(
            t_hbm.at[i], buf_t.at[slot], sem_t.at[slot]).start()
        pltpu.make_async_copy(
            p_hbm.at[i], buf_p.at[slot], sem_p.at[slot]).start()

    def wait_in(slot):
        pltpu.make_async_copy(
            t_hbm.at[0], buf_t.at[slot], sem_t.at[slot]).wait()
        pltpu.make_async_copy(
            p_hbm.at[0], buf_p.at[slot], sem_p.at[slot]).wait()

    dma_in(0, 0)

    def body(i, _):
        cur = jax.lax.rem(i, 2)
        nxt = jax.lax.rem(i + 1, 2)

        @pl.when(i + 1 < nb)
        def _():
            dma_in(nxt, i + 1)

        wait_in(cur)
        _one_batch(buf_t.at[cur], buf_p.at[cur], i, out_ref,
                   hist_p, hist_t, zt_ref, zp_ref)
        return ()

    jax.lax.fori_loop(0, nb, body, ())


def _kl_pallas(p3, t3):
    b, rows, lanes = p3.shape
    return pl.pallas_call(
        _kl_body,
        out_shape=jax.ShapeDtypeStruct((b, 8, 128), jnp.float32),
        in_specs=[
            pl.BlockSpec(memory_space=pl.ANY),
            pl.BlockSpec(memory_space=pl.ANY),
        ],
        scratch_shapes=[
            pltpu.VMEM((2, rows, lanes), jnp.float32),
            pltpu.VMEM((2, rows, lanes), jnp.float32),
            pltpu.VMEM((_NBINS, 128), jnp.float32),
            pltpu.VMEM((_NBINS, 128), jnp.float32),
            pltpu.VMEM((rows, lanes), jnp.float32),
            pltpu.VMEM((rows, lanes), jnp.float32),
            pltpu.SemaphoreType.DMA((2,)),
            pltpu.SemaphoreType.DMA((2,)),
        ],
        name="kl_soft_hist",
    )(p3, t3)


def kernel(pred, target):
    B = pred.shape[0]
    n = pred.size // B
    lanes = 128
    rows = n // lanes
    p3 = pred.reshape(B, rows, lanes)
    t3 = target.reshape(B, rows, lanes)

    out = _kl_pallas(p3, t3)

    return _W * jnp.mean(out[:, 0, 0])


# speedup vs baseline: 1.0159x; 1.0159x over previous
import jax
import jax.numpy as jnp
from jax.experimental import pallas as pl
from jax.experimental.pallas import tpu as pltpu
import math

_W = 0.1
_NBINS = 64
_EPS = 1e-08
_LOG2E = 1.4426950408889634
_K = 8
_CH = 64


def _sum8(v):
    return jnp.sum(v.reshape(_CH // 8, 8, 128), axis=0)


def _one_batch(t_ref, p_ref, i, out_ref, hist_p, hist_t, zt_ref, zp_ref):
    rows = t_ref.shape[0]
    nch = rows // _CH

    def mm_body(c, carry):
        mn, mx = carry
        tc = t_ref[pl.ds(c * _CH, _CH), :].reshape(_CH // 8, 8, 128)
        return (jnp.minimum(mn, jnp.min(tc, axis=0)),
                jnp.maximum(mx, jnp.max(tc, axis=0)))

    mn0 = jnp.full((8, 128), jnp.inf, jnp.float32)
    mx0 = jnp.full((8, 128), -jnp.inf, jnp.float32)
    mn, mx = jax.lax.fori_loop(0, nch, mm_body, (mn0, mx0), unroll=4)
    vmin = jnp.min(mn)
    vmax = jnp.max(mx)
    denom = vmax - vmin + _EPS
    ia = _NBINS / denom
    zt_ref[...] = (t_ref[...] - vmin) * ia
    zp_ref[...] = (p_ref[...] - vmin) * ia

    nL2 = jnp.float32(-_LOG2E / 2.0)
    L = jnp.float32(_LOG2E)
    G = _NBINS // _K
    scales = [jnp.float32(math.exp(-(k * k) / 2.0)) for k in range(_K)]

    for g in range(G):
        sh = jnp.float32(g * _K + 0.5)
        hi = jnp.float32(_K + 39.0)

        def chunk_body(c, accs, z_ref=None):
            zc = z_ref[pl.ds(c * _CH, _CH), :]
            u = jnp.minimum(zc - sh, hi)
            S = jnp.exp2((u * nL2) * u)
            r = jnp.exp2(u * L)
            out = []
            for k in range(_K):
                if k:
                    S = S * r
                out.append(accs[k] + _sum8(S))
            return tuple(out)

        zero = tuple(jnp.zeros((8, 128), jnp.float32) for _ in range(_K))

        acc_t = jax.lax.fori_loop(
            0, nch, lambda c, a: chunk_body(c, a, z_ref=zt_ref), zero,
            unroll=8)
        acc_p = jax.lax.fori_loop(
            0, nch, lambda c, a: chunk_body(c, a, z_ref=zp_ref), zero,
            unroll=8)
        for k in range(_K):
            b = g * _K + k
            hist_t[b:b + 1, :] = scales[k] * jnp.sum(
                acc_t[k], axis=0, keepdims=True)
            hist_p[b:b + 1, :] = scales[k] * jnp.sum(
                acc_p[k], axis=0, keepdims=True)

    ht = jnp.sum(hist_t[...], axis=1, keepdims=True)
    hp = jnp.sum(hist_p[...], axis=1, keepdims=True)
    tp = ht / (jnp.sum(ht) + _EPS)
    pp = hp / (jnp.sum(hp) + _EPS)
    kl = jnp.sum(tp * (jnp.log(tp + _EPS) - jnp.log(pp + _EPS)))
    out_ref[i] = jnp.full((8, 128), kl, dtype=jnp.float32)


def _kl_body(p_hbm, t_hbm, out_ref,
             buf_t, buf_p, hist_p, hist_t, zt_ref, zp_ref, sem_t, sem_p):
    nb = p_hbm.shape[0]

    def dma_in(slot, i):
        pltpu.make_async_copy(
            t_hbm.at[i], buf_t.at[slot], sem_t.at[slot]).start()
        pltpu.make_async_copy(
            p_hbm.at[i], buf_p.at[slot], sem_p.at[slot]).start()

    def wait_in(slot):
        pltpu.make_async_copy(
            t_hbm.at[0], buf_t.at[slot], sem_t.at[slot]).wait()
        pltpu.make_async_copy(
            p_hbm.at[0], buf_p.at[slot], sem_p.at[slot]).wait()

    dma_in(0, 0)

    def body(i, _):
        cur = jax.lax.rem(i, 2)
        nxt = jax.lax.rem(i + 1, 2)

        @pl.when(i + 1 < nb)
        def _():
            dma_in(nxt, i + 1)

        wait_in(cur)
        _one_batch(buf_t.at[cur], buf_p.at[cur], i, out_ref,
                   hist_p, hist_t, zt_ref, zp_ref)
        return ()

    jax.lax.fori_loop(0, nb, body, ())


def _kl_pallas(p3, t3):
    b, rows, lanes = p3.shape
    return pl.pallas_call(
        _kl_body,
        out_shape=jax.ShapeDtypeStruct((b, 8, 128), jnp.float32),
        in_specs=[
            pl.BlockSpec(memory_space=pl.ANY),
            pl.BlockSpec(memory_space=pl.ANY),
        ],
        scratch_shapes=[
            pltpu.VMEM((2, rows, lanes), jnp.float32),
            pltpu.VMEM((2, rows, lanes), jnp.float32),
            pltpu.VMEM((_NBINS, 128), jnp.float32),
            pltpu.VMEM((_NBINS, 128), jnp.float32),
            pltpu.VMEM((rows, lanes), jnp.float32),
            pltpu.VMEM((rows, lanes), jnp.float32),
            pltpu.SemaphoreType.DMA((2,)),
            pltpu.SemaphoreType.DMA((2,)),
        ],
        name="kl_soft_hist",
    )(p3, t3)


def kernel(pred, target):
    B = pred.shape[0]
    n = pred.size // B
    lanes = 128
    rows = n // lanes
    p3 = pred.reshape(B, rows, lanes)
    t3 = target.reshape(B, rows, lanes)

    out = _kl_pallas(p3, t3)

    return _W * jnp.mean(out[:, 0, 0])


# speedup vs baseline: 1.0305x; 1.0143x over previous
import jax
import jax.numpy as jnp
from jax.experimental import pallas as pl
from jax.experimental.pallas import tpu as pltpu
import math

_W = 0.1
_NBINS = 64
_EPS = 1e-08
_LOG2E = 1.4426950408889634
_K = 8
_CH = 64


def _sum8(v):
    return jnp.sum(v.reshape(_CH // 8, 8, 128), axis=0)


def _one_batch(t_ref, p_ref, i, out_ref, hist_p, hist_t, zt_ref, zp_ref):
    rows = t_ref.shape[0]
    nch = rows // _CH

    def mm_body(c, carry):
        mn, mx = carry
        tc = t_ref[pl.ds(c * _CH, _CH), :].reshape(_CH // 8, 8, 128)
        return (jnp.minimum(mn, jnp.min(tc, axis=0)),
                jnp.maximum(mx, jnp.max(tc, axis=0)))

    mn0 = jnp.full((8, 128), jnp.inf, jnp.float32)
    mx0 = jnp.full((8, 128), -jnp.inf, jnp.float32)
    mn, mx = jax.lax.fori_loop(0, nch, mm_body, (mn0, mx0), unroll=4)
    vmin = jnp.min(mn)
    vmax = jnp.max(mx)
    denom = vmax - vmin + _EPS
    ia = _NBINS / denom
    zt_ref[...] = (t_ref[...] - vmin) * ia
    zp_ref[...] = (p_ref[...] - vmin) * ia

    nL2 = jnp.float32(-_LOG2E / 2.0)
    L = jnp.float32(_LOG2E)
    G = _NBINS // _K
    scales = [jnp.float32(math.exp(-(k * k) / 2.0)) for k in range(_K)]

    for g in range(G):
        sh = jnp.float32(g * _K + 0.5)
        hi = jnp.float32(_K + 39.0)

        def chunk_body(c, accs, z_ref=None):
            zc = z_ref[pl.ds(c * _CH, _CH), :]
            u = jnp.minimum(zc - sh, hi)
            S = jnp.exp2((u * nL2) * u)
            r = jnp.exp2(u * L)
            out = []
            for k in range(_K):
                if k:
                    S = S * r
                out.append(accs[k] + _sum8(S))
            return tuple(out)

        zero = tuple(jnp.zeros((8, 128), jnp.float32) for _ in range(_K))

        acc_t = jax.lax.fori_loop(
            0, nch, lambda c, a: chunk_body(c, a, z_ref=zt_ref), zero,
            unroll=16)
        acc_p = jax.lax.fori_loop(
            0, nch, lambda c, a: chunk_body(c, a, z_ref=zp_ref), zero,
            unroll=16)
        for k in range(_K):
            b = g * _K + k
            hist_t[b:b + 1, :] = scales[k] * jnp.sum(
                acc_t[k], axis=0, keepdims=True)
            hist_p[b:b + 1, :] = scales[k] * jnp.sum(
                acc_p[k], axis=0, keepdims=True)

    ht = jnp.sum(hist_t[...], axis=1, keepdims=True)
    hp = jnp.sum(hist_p[...], axis=1, keepdims=True)
    tp = ht / (jnp.sum(ht) + _EPS)
    pp = hp / (jnp.sum(hp) + _EPS)
    kl = jnp.sum(tp * (jnp.log(tp + _EPS) - jnp.log(pp + _EPS)))
    out_ref[i] = jnp.full((8, 128), kl, dtype=jnp.float32)


def _kl_body(p_hbm, t_hbm, out_ref,
             buf_t, buf_p, hist_p, hist_t, zt_ref, zp_ref, sem_t, sem_p):
    nb = p_hbm.shape[0]

    def dma_in(slot, i):
        pltpu.make_async_copy(
            t_hbm.at[i], buf_t.at[slot], sem_t.at[slot]).start()
        pltpu.make_async_copy(
            p_hbm.at[i], buf_p.at[slot], sem_p.at[slot]).start()

    def wait_in(slot):
        pltpu.make_async_copy(
            t_hbm.at[0], buf_t.at[slot], sem_t.at[slot]).wait()
        pltpu.make_async_copy(
            p_hbm.at[0], buf_p.at[slot], sem_p.at[slot]).wait()

    dma_in(0, 0)

    def body(i, _):
        cur = jax.lax.rem(i, 2)
        nxt = jax.lax.rem(i + 1, 2)

        @pl.when(i + 1 < nb)
        def _():
            dma_in(nxt, i + 1)

        wait_in(cur)
        _one_batch(buf_t.at[cur], buf_p.at[cur], i, out_ref,
                   hist_p, hist_t, zt_ref, zp_ref)
        return ()

    jax.lax.fori_loop(0, nb, body, ())


def _kl_pallas(p3, t3):
    b, rows, lanes = p3.shape
    return pl.pallas_call(
        _kl_body,
        out_shape=jax.ShapeDtypeStruct((b, 8, 128), jnp.float32),
        in_specs=[
            pl.BlockSpec(memory_space=pl.ANY),
            pl.BlockSpec(memory_space=pl.ANY),
        ],
        scratch_shapes=[
            pltpu.VMEM((2, rows, lanes), jnp.float32),
            pltpu.VMEM((2, rows, lanes), jnp.float32),
            pltpu.VMEM((_NBINS, 128), jnp.float32),
            pltpu.VMEM((_NBINS, 128), jnp.float32),
            pltpu.VMEM((rows, lanes), jnp.float32),
            pltpu.VMEM((rows, lanes), jnp.float32),
            pltpu.SemaphoreType.DMA((2,)),
            pltpu.SemaphoreType.DMA((2,)),
        ],
        name="kl_soft_hist",
    )(p3, t3)


def kernel(pred, target):
    B = pred.shape[0]
    n = pred.size // B
    lanes = 128
    rows = n // lanes
    p3 = pred.reshape(B, rows, lanes)
    t3 = target.reshape(B, rows, lanes)

    out = _kl_pallas(p3, t3)

    return _W * jnp.mean(out[:, 0, 0])
